# baseline (speedup 1.0000x reference)
import functools

import jax
import jax.numpy as jnp
from jax import lax
from jax.experimental import pallas as pl
from jax.experimental.pallas import tpu as pltpu
from jax.experimental.pallas import tpu_sc as plsc

B = 16384
C = 5
V = 1000000
D = 64

NC = 2
NS = 16
NW = NC * NS
BPW = B // NW
CB = 256
NCHUNK = BPW // CB
XROWS = CB * C
IDXW = 128
TIDX_R = BPW // IDXW
XIDX_R = BPW * C // IDXW
TIDX_C = CB // IDXW
XIDX_C = XROWS // IDXW
DV = D // 16
BG = 16
NOUTV = BG * C // 16


def _sc_body(tgt_idx_hbm, ctx_idx_hbm, tgt_tab_hbm, ctx_tab_hbm, out_hbm,
             tidx_v, xidx_v, trows_v, xrows_v, out_v, sem):
    w = lax.axis_index("s") * NC + lax.axis_index("c")
    lane = lax.iota(jnp.int32, 16)
    pltpu.sync_copy(tgt_idx_hbm.at[w], tidx_v)
    pltpu.sync_copy(ctx_idx_hbm.at[w], xidx_v)
    for q in range(NCHUNK):
        base = w * BPW + q * CB
        copies = []
        for j in range(TIDX_C):
            copies.append(pltpu.async_copy(
                tgt_tab_hbm.at[tidx_v.at[q * TIDX_C + j]],
                trows_v.at[pl.ds(j * IDXW, IDXW)], sem))
        for j in range(XIDX_C):
            copies.append(pltpu.async_copy(
                ctx_tab_hbm.at[xidx_v.at[q * XIDX_C + j]],
                xrows_v.at[pl.ds(j * IDXW, IDXW)], sem))
        for cp in copies:
            cp.wait()

        def body(g, carry):
            rx = g * 16 + lane
            rt = rx // C
            acc = jnp.zeros((16,), jnp.float32)
            for d in range(D):
                dvec = jnp.full((16,), d, jnp.int32)
                tv = plsc.load_gather(trows_v, [rt, dvec])
                xv = plsc.load_gather(xrows_v, [rx, dvec])
                acc = acc + tv * xv
            out_v[pl.ds(g * 16, 16)] = acc
            return carry
        lax.fori_loop(0, CB * C // 16, body, 0)
        pltpu.sync_copy(out_v, out_hbm.at[pl.ds(base * C, CB * C)])


_mesh = plsc.VectorSubcoreMesh(core_axis_name="c", subcore_axis_name="s")

_skipgram_sc = functools.partial(
    pl.kernel,
    out_type=jax.ShapeDtypeStruct((B * C,), jnp.float32),
    mesh=_mesh,
    scratch_types=[
        pltpu.VMEM((TIDX_R, IDXW), jnp.int32),
        pltpu.VMEM((XIDX_R, IDXW), jnp.int32),
        pltpu.VMEM((CB, D), jnp.float32),
        pltpu.VMEM((XROWS, D), jnp.float32),
        pltpu.VMEM((CB * C,), jnp.float32),
        pltpu.SemaphoreType.DMA,
    ],
    compiler_params=pltpu.CompilerParams(
        needs_layout_passes=False, use_tc_tiling_on_sc=False),
)(_sc_body)


def kernel(target, context, skipgram_table, context_table):
    tgt3d = target.astype(jnp.int32).reshape(NW, TIDX_R, IDXW)
    ctx3d = context.astype(jnp.int32).reshape(NW, XIDX_R, IDXW)
    return _skipgram_sc(tgt3d, ctx3d, skipgram_table, context_table).reshape(B, C)
